# baseline (speedup 1.0000x reference)
import jax
import jax.numpy as jnp
from jax import lax
from jax.experimental import pallas as pl
from jax.experimental.pallas import tpu as pltpu
from jax.experimental.pallas import tpu_sc as plsc

D_MODEL = 128
BATCH = 132
NUM_TABLES = 6
LANES = 16
FULL_TILES = BATCH // LANES
TAIL_ROWS = BATCH - FULL_TILES * LANES


def _sc_body(xf_hbm, t0, t1, t2, t3, t4, t5, out_hbm, xblk_v, idx_v, gath_v,
             acc_v, sem):
    cid = lax.axis_index("c")
    sid = lax.axis_index("s")
    wid = sid * 2 + cid
    tables = (t0, t1, t2, t3, t4, t5)

    def work(base_row, nrows):
        nelem = nrows * NUM_TABLES
        pltpu.sync_copy(
            xf_hbm.at[pl.ds(base_row * NUM_TABLES, nelem)],
            xblk_v.at[pl.ds(0, nelem)],
        )
        lane = lax.iota(jnp.int32, LANES)
        mask = lane < nrows
        for t in range(NUM_TABLES):
            gidx = lane * NUM_TABLES + t
            if nrows < LANES:
                v = plsc.load_gather(xblk_v, [gidx], mask=mask)
                v = jnp.where(mask, v, 0)
            else:
                v = plsc.load_gather(xblk_v, [gidx])
            idx_v[t, :] = v
        copies = []
        for t in range(NUM_TABLES):
            copies.append(
                pltpu.async_copy(tables[t].at[idx_v.at[t]], gath_v.at[t], sem)
            )
        for cp in copies:
            cp.wait()
        for i in range(nrows):
            for c in range(D_MODEL // LANES):
                sl = pl.ds(c * LANES, LANES)
                acc_v[i, sl] = (
                    gath_v[0, i, sl]
                    + gath_v[1, i, sl]
                    + gath_v[2, i, sl]
                    + gath_v[3, i, sl]
                    + gath_v[4, i, sl]
                    + gath_v[5, i, sl]
                )
        pltpu.sync_copy(
            acc_v.at[pl.ds(0, nrows)], out_hbm.at[pl.ds(base_row, nrows)]
        )

    @pl.when(wid < FULL_TILES)
    def _():
        work(wid * LANES, LANES)

    @pl.when(wid == FULL_TILES)
    def _():
        work(FULL_TILES * LANES, TAIL_ROWS)


@jax.jit
def _sc_embed(xf, turn_table, card_table, action_table, pos_table, civ_table,
              face_table):
    mesh = plsc.VectorSubcoreMesh(core_axis_name="c", subcore_axis_name="s")
    return pl.kernel(
        _sc_body,
        out_type=jax.ShapeDtypeStruct((BATCH, D_MODEL), jnp.float32),
        mesh=mesh,
        scratch_types=[
            pltpu.VMEM((LANES * NUM_TABLES,), jnp.int32),
            pltpu.VMEM((NUM_TABLES, LANES), jnp.int32),
            pltpu.VMEM((NUM_TABLES, LANES, D_MODEL), jnp.float32),
            pltpu.VMEM((LANES, D_MODEL), jnp.float32),
            pltpu.SemaphoreType.DMA,
        ],
        compiler_params=pltpu.CompilerParams(needs_layout_passes=False),
    )(xf, turn_table, card_table, action_table, pos_table, civ_table,
      face_table)


def kernel(x, turn_table, card_table, action_table, pos_table, civ_table,
           face_table):
    xf = jnp.reshape(x.astype(jnp.int32), (-1,))
    return _sc_embed(xf, turn_table, card_table, action_table, pos_table,
                     civ_table, face_table)


# speedup vs baseline: 1.1790x; 1.1790x over previous
import jax
import jax.numpy as jnp
from jax import lax
from jax.experimental import pallas as pl
from jax.experimental.pallas import tpu as pltpu
from jax.experimental.pallas import tpu_sc as plsc

D_MODEL = 128
BATCH = 132
NUM_TABLES = 6
LANES = 16
NUM_TILES = 9
LAST_BASE = BATCH - LANES


def _sc_body(xf_hbm, t0, t1, t2, t3, t4, t5, out_hbm, xblk_v, idx_v, gath_v,
             acc_v, sem):
    cid = lax.axis_index("c")
    sid = lax.axis_index("s")
    wid = sid * 2 + cid
    tables = (t0, t1, t2, t3, t4, t5)

    @pl.when(wid < NUM_TILES)
    def _():
        base_row = jnp.minimum(wid * LANES, LAST_BASE)
        pltpu.sync_copy(
            xf_hbm.at[pl.ds(base_row * NUM_TABLES, LANES * NUM_TABLES)],
            xblk_v,
        )
        lane = lax.iota(jnp.int32, LANES)
        for t in range(NUM_TABLES):
            idx_v[t, :] = plsc.load_gather(xblk_v, [lane * NUM_TABLES + t])
        copies = []
        for t in range(NUM_TABLES):
            copies.append(
                pltpu.async_copy(tables[t].at[idx_v.at[t]], gath_v.at[t], sem)
            )
        for cp in copies:
            cp.wait()

        def row(i, _):
            for c in range(D_MODEL // LANES):
                sl = pl.ds(c * LANES, LANES)
                acc_v[i, sl] = (
                    gath_v[0, i, sl]
                    + gath_v[1, i, sl]
                    + gath_v[2, i, sl]
                    + gath_v[3, i, sl]
                    + gath_v[4, i, sl]
                    + gath_v[5, i, sl]
                )
            return 0

        lax.fori_loop(0, LANES, row, 0)

        @pl.when(wid < NUM_TILES - 1)
        def _():
            off = pl.multiple_of(wid * LANES, 8)
            pltpu.sync_copy(acc_v, out_hbm.at[pl.ds(off, LANES)])

        @pl.when(wid == NUM_TILES - 1)
        def _():
            pltpu.sync_copy(
                acc_v.at[pl.ds(128 - LAST_BASE, BATCH - 128)],
                out_hbm.at[pl.ds(128, BATCH - 128)],
            )


@jax.jit
def _sc_embed(xf, turn_table, card_table, action_table, pos_table, civ_table,
              face_table):
    mesh = plsc.VectorSubcoreMesh(core_axis_name="c", subcore_axis_name="s")
    return pl.kernel(
        _sc_body,
        out_type=jax.ShapeDtypeStruct((BATCH, D_MODEL), jnp.float32),
        mesh=mesh,
        scratch_types=[
            pltpu.VMEM((LANES * NUM_TABLES,), jnp.int32),
            pltpu.VMEM((NUM_TABLES, LANES), jnp.int32),
            pltpu.VMEM((NUM_TABLES, LANES, D_MODEL), jnp.float32),
            pltpu.VMEM((LANES, D_MODEL), jnp.float32),
            pltpu.SemaphoreType.DMA,
        ],
        compiler_params=pltpu.CompilerParams(needs_layout_passes=False),
    )(xf, turn_table, card_table, action_table, pos_table, civ_table,
      face_table)


def kernel(x, turn_table, card_table, action_table, pos_table, civ_table,
           face_table):
    xf = jnp.reshape(x.astype(jnp.int32), (-1,))
    return _sc_embed(xf, turn_table, card_table, action_table, pos_table,
                     civ_table, face_table)


# speedup vs baseline: 1.2352x; 1.0477x over previous
import jax
import jax.numpy as jnp
from jax import lax
from jax.experimental import pallas as pl
from jax.experimental.pallas import tpu as pltpu
from jax.experimental.pallas import tpu_sc as plsc

D_MODEL = 128
BATCH = 132
NUM_TABLES = 6
LANES = 16
NUM_TILES = 9
LAST_BASE = BATCH - LANES


def _sc_body(xf_hbm, t0, t1, t2, t3, t4, t5, out_hbm, xblk_v, idx_v, gath_v,
             acc_v, sem):
    wid = lax.axis_index("s")
    tables = (t0, t1, t2, t3, t4, t5)

    @pl.when(wid < NUM_TILES)
    def _():
        base_row = jnp.minimum(wid * LANES, LAST_BASE)
        pltpu.sync_copy(
            xf_hbm.at[pl.ds(base_row * NUM_TABLES, LANES * NUM_TABLES)],
            xblk_v,
        )
        lane = lax.iota(jnp.int32, LANES)
        for t in range(NUM_TABLES):
            idx_v[t, :] = plsc.load_gather(xblk_v, [lane * NUM_TABLES + t])
        copies = []
        for t in range(NUM_TABLES):
            copies.append(
                pltpu.async_copy(tables[t].at[idx_v.at[t]], gath_v.at[t], sem)
            )
        for cp in copies:
            cp.wait()

        def row(i, _):
            for c in range(D_MODEL // LANES):
                sl = pl.ds(c * LANES, LANES)
                acc_v[i, sl] = (
                    gath_v[0, i, sl]
                    + gath_v[1, i, sl]
                    + gath_v[2, i, sl]
                    + gath_v[3, i, sl]
                    + gath_v[4, i, sl]
                    + gath_v[5, i, sl]
                )
            return 0

        lax.fori_loop(0, LANES, row, 0)

        @pl.when(wid < NUM_TILES - 1)
        def _():
            off = pl.multiple_of(wid * LANES, 8)
            pltpu.sync_copy(acc_v, out_hbm.at[pl.ds(off, LANES)])

        @pl.when(wid == NUM_TILES - 1)
        def _():
            pltpu.sync_copy(
                acc_v.at[pl.ds(128 - LAST_BASE, BATCH - 128)],
                out_hbm.at[pl.ds(128, BATCH - 128)],
            )


@jax.jit
def _sc_embed(xf, turn_table, card_table, action_table, pos_table, civ_table,
              face_table):
    mesh = plsc.VectorSubcoreMesh(core_axis_name="c", subcore_axis_name="s",
                                  num_cores=1)
    return pl.kernel(
        _sc_body,
        out_type=jax.ShapeDtypeStruct((BATCH, D_MODEL), jnp.float32),
        mesh=mesh,
        scratch_types=[
            pltpu.VMEM((LANES * NUM_TABLES,), jnp.int32),
            pltpu.VMEM((NUM_TABLES, LANES), jnp.int32),
            pltpu.VMEM((NUM_TABLES, LANES, D_MODEL), jnp.float32),
            pltpu.VMEM((LANES, D_MODEL), jnp.float32),
            pltpu.SemaphoreType.DMA,
        ],
        compiler_params=pltpu.CompilerParams(needs_layout_passes=False),
    )(xf, turn_table, card_table, action_table, pos_table, civ_table,
      face_table)


def kernel(x, turn_table, card_table, action_table, pos_table, civ_table,
           face_table):
    xf = jnp.reshape(x.astype(jnp.int32), (-1,))
    return _sc_embed(xf, turn_table, card_table, action_table, pos_table,
                     civ_table, face_table)
